# baseline (speedup 1.0000x reference)
from functools import partial

import numpy as np
import jax
import jax.numpy as jnp
from jax import lax
from jax.experimental import pallas as pl
from jax.experimental.pallas import tpu as pltpu


def _inception_kernel(x_ref, fw_ref, fs_ref, ff_ref, b4s_ref,
                      w2_ref, s2_ref, w32_ref, s32_ref, w33_ref, s33_ref,
                      o_ref, *, H, W):
    HW = H * W
    C = 128

    xb = x_ref[0]
    fused = lax.dot_general(xb, fw_ref[...], (((0,), (0,)), ((), ())),
                            preferred_element_type=jnp.float32)
    fb = jnp.maximum(fused + fs_ref[...], ff_ref[...]).astype(jnp.bfloat16)

    def conv3(src, w_ref, s_ref, out_bf16):
        x3 = src.reshape(H, W, C)
        zr = jnp.zeros((1, W, C), jnp.bfloat16)
        xv = jnp.concatenate([zr, x3, zr], axis=0)
        rows3 = jnp.concatenate([xv[0:H], xv[1:H + 1], xv[2:H + 2]],
                                axis=-1).reshape(HW, 3 * C)
        z = jnp.dot(rows3, w_ref[...], preferred_element_type=jnp.float32)
        z0 = z[:, 0:C].reshape(H, W, C)
        z1 = z[:, C:2 * C].reshape(H, W, C)
        z2 = z[:, 2 * C:3 * C].reshape(H, W, C)
        zc = jnp.zeros((H, 1, C), jnp.float32)
        y = (z1 + jnp.concatenate([zc, z0[:, 0:W - 1]], axis=1)
             + jnp.concatenate([z2[:, 1:W], zc], axis=1)).reshape(HW, C)
        y = jnp.maximum(y + s_ref[...], 0.0)
        return y.astype(jnp.bfloat16) if out_bf16 else y

    x2 = conv3(fb[:, C:2 * C], w2_ref, s2_ref, False)
    t3 = conv3(fb[:, 2 * C:3 * C], w32_ref, s32_ref, True)
    x3 = conv3(t3, w33_ref, s33_ref, False)

    f4 = fb[:, 3 * C:4 * C].astype(jnp.float32).reshape(H, W, C)
    zr = jnp.zeros((1, W, C), jnp.float32)
    xv = jnp.concatenate([zr, f4, zr], axis=0)
    rows = xv[0:H] + xv[1:H + 1] + xv[2:H + 2]
    zc = jnp.zeros((H, 1, C), jnp.float32)
    rp = jnp.concatenate([zc, rows, zc], axis=1)
    x4 = jnp.maximum((rp[:, 0:W] + rp[:, 1:W + 1] + rp[:, 2:W + 2])
                     .reshape(HW, C) + b4s_ref[...], 0.0)

    x1 = fb[:, 0:C].astype(jnp.float32)
    o_ref[0] = jnp.concatenate(
        [jnp.transpose(x1)[0:96], jnp.transpose(x2)[0:96],
         jnp.transpose(x3)[0:96], jnp.transpose(x4)[0:96]], axis=0)


def kernel(x_nchw, fused_w, fused_s, fused_floor, b4_s,
           b2_2_w, b2_2_s, b3_2_w, b3_2_s, b3_3_w, b3_3_s):
    N, Cin, H, W = x_nchw.shape
    HW = H * W
    x = x_nchw.reshape(N, Cin, HW).astype(jnp.bfloat16)
    Cout = fused_w.shape[1]

    def _retap(w):
        C = w.shape[1]
        return w.reshape(3, 3, C, C).transpose(0, 2, 1, 3).reshape(3 * C, 3 * C)

    w2 = _retap(b2_2_w)
    w32 = _retap(b3_2_w)
    w33 = _retap(b3_3_w)

    est = (2 * Cin * HW * 4
           + 2 * 384 * HW * 4
           + Cin * Cout * 2 + 3 * 1152 * 128 * 2
           + HW * Cout * 4 + HW * Cout * 2
           + HW * 9 * 128 * 2
           + 6 * HW * 128 * 4)
    limit = 30 << 20

    out = pl.pallas_call(
        partial(_inception_kernel, H=H, W=W),
        out_shape=jax.ShapeDtypeStruct((N, 384, HW), jnp.float32),
        grid=(N,),
        in_specs=[
            pl.BlockSpec((1, Cin, HW), lambda n: (n, 0, 0)),
            pl.BlockSpec((Cin, Cout), lambda n: (0, 0)),
            pl.BlockSpec((1, Cout), lambda n: (0, 0)),
            pl.BlockSpec((1, Cout), lambda n: (0, 0)),
            pl.BlockSpec((1, 128), lambda n: (0, 0)),
            pl.BlockSpec((384, 384), lambda n: (0, 0)),
            pl.BlockSpec((1, 128), lambda n: (0, 0)),
            pl.BlockSpec((384, 384), lambda n: (0, 0)),
            pl.BlockSpec((1, 128), lambda n: (0, 0)),
            pl.BlockSpec((384, 384), lambda n: (0, 0)),
            pl.BlockSpec((1, 128), lambda n: (0, 0)),
        ],
        out_specs=pl.BlockSpec((1, 384, HW), lambda n: (n, 0, 0)),
        compiler_params=pltpu.CompilerParams(
            dimension_semantics=("parallel",),
            vmem_limit_bytes=limit),
    )(x, fused_w, fused_s, fused_floor, b4_s,
      w2, b2_2_s, w32, b3_2_s, w33, b3_3_s)
    return out.reshape(N, 384, H, W)


# speedup vs baseline: 1.1049x; 1.1049x over previous
from functools import partial

import jax
import jax.numpy as jnp
from jax import lax
from jax.experimental import pallas as pl
from jax.experimental.pallas import tpu as pltpu


def _inception_kernel(x_ref, fw_ref, fs_ref, ff_ref, b4s_ref,
                      w2_ref, s2_ref, w32_ref, s32_ref, w33_ref, s33_ref,
                      o1_ref, o2_ref, o3_ref, o4_ref, *, H, W):
    HW = H * W
    C = 128

    xb = x_ref[0].astype(jnp.bfloat16)
    fused = lax.dot_general(xb, fw_ref[...], (((0,), (0,)), ((), ())),
                            preferred_element_type=jnp.float32)
    fb = jnp.maximum(fused + fs_ref[...], ff_ref[...]).astype(jnp.bfloat16)

    def conv3(src, w_ref, s_ref):
        x3 = src.reshape(H, W, C)
        zr = jnp.zeros((1, W, C), jnp.bfloat16)
        xv = jnp.concatenate([zr, x3, zr], axis=0)
        rows3 = jnp.concatenate([xv[0:H], xv[1:H + 1], xv[2:H + 2]],
                                axis=-1).reshape(HW, 3 * C)
        z = jnp.dot(rows3, w_ref[...], preferred_element_type=jnp.float32)
        z0 = z[:, 0:C].reshape(H, W, C)
        z1 = z[:, C:2 * C].reshape(H, W, C)
        z2 = z[:, 2 * C:3 * C].reshape(H, W, C)
        zc = jnp.zeros((H, 1, C), jnp.float32)
        y = (z1 + jnp.concatenate([zc, z0[:, 0:W - 1]], axis=1)
             + jnp.concatenate([z2[:, 1:W], zc], axis=1)).reshape(HW, C)
        return jnp.maximum(y + s_ref[...], 0.0)

    x2 = conv3(fb[:, C:2 * C], w2_ref, s2_ref)
    t3 = conv3(fb[:, 2 * C:3 * C], w32_ref, s32_ref).astype(jnp.bfloat16)
    x3 = conv3(t3, w33_ref, s33_ref)

    f4 = fb[:, 3 * C:4 * C].astype(jnp.float32).reshape(H, W, C)
    zr = jnp.zeros((1, W, C), jnp.float32)
    xv = jnp.concatenate([zr, f4, zr], axis=0)
    rows = xv[0:H] + xv[1:H + 1] + xv[2:H + 2]
    zc = jnp.zeros((H, 1, C), jnp.float32)
    rp = jnp.concatenate([zc, rows, zc], axis=1)
    x4 = jnp.maximum((rp[:, 0:W] + rp[:, 1:W + 1] + rp[:, 2:W + 2])
                     .reshape(HW, C) + b4s_ref[...], 0.0)

    o1_ref[0] = fb[:, 0:C]
    o2_ref[0] = x2.astype(jnp.bfloat16)
    o3_ref[0] = x3.astype(jnp.bfloat16)
    o4_ref[0] = x4.astype(jnp.bfloat16)


def kernel(x_nchw, fused_w, fused_s, fused_floor, b4_s,
           b2_2_w, b2_2_s, b3_2_w, b3_2_s, b3_3_w, b3_3_s):
    N, Cin, H, W = x_nchw.shape
    HW = H * W
    x = x_nchw.reshape(N, Cin, HW)
    Cout = fused_w.shape[1]

    def _retap(w):
        C = w.shape[1]
        return w.reshape(3, 3, C, C).transpose(0, 2, 1, 3).reshape(3 * C, 3 * C)

    w2 = _retap(b2_2_w)
    w32 = _retap(b3_2_w)
    w33 = _retap(b3_3_w)

    obs = pl.BlockSpec((1, HW, 128), lambda n: (n, 0, 0))
    osh = jax.ShapeDtypeStruct((N, HW, 128), jnp.bfloat16)
    o1, o2, o3, o4 = pl.pallas_call(
        partial(_inception_kernel, H=H, W=W),
        out_shape=(osh, osh, osh, osh),
        grid=(N,),
        in_specs=[
            pl.BlockSpec((1, Cin, HW), lambda n: (n, 0, 0)),
            pl.BlockSpec((Cin, Cout), lambda n: (0, 0)),
            pl.BlockSpec((1, Cout), lambda n: (0, 0)),
            pl.BlockSpec((1, Cout), lambda n: (0, 0)),
            pl.BlockSpec((1, 128), lambda n: (0, 0)),
            pl.BlockSpec((384, 384), lambda n: (0, 0)),
            pl.BlockSpec((1, 128), lambda n: (0, 0)),
            pl.BlockSpec((384, 384), lambda n: (0, 0)),
            pl.BlockSpec((1, 128), lambda n: (0, 0)),
            pl.BlockSpec((384, 384), lambda n: (0, 0)),
            pl.BlockSpec((1, 128), lambda n: (0, 0)),
        ],
        out_specs=(obs, obs, obs, obs),
        compiler_params=pltpu.CompilerParams(
            dimension_semantics=("parallel",),
            vmem_limit_bytes=24 << 20),
    )(x, fused_w, fused_s, fused_floor, b4_s,
      w2, b2_2_s, w32, b3_2_s, w33, b3_3_s)

    out = jnp.concatenate([o1[:, :, 0:96], o2[:, :, 0:96],
                           o3[:, :, 0:96], o4[:, :, 0:96]],
                          axis=-1).astype(jnp.float32)
    return jnp.transpose(out.reshape(N, H, W, 384), (0, 3, 1, 2))


# speedup vs baseline: 1.1275x; 1.0204x over previous
from functools import partial

import jax
import jax.numpy as jnp
from jax import lax
from jax.experimental import pallas as pl
from jax.experimental.pallas import tpu as pltpu


def _inception_kernel(x_ref, fw_ref, fs_ref, ff_ref, b4s_ref,
                      w2_ref, s2_ref, w32_ref, s32_ref, w33_ref, s33_ref,
                      o1_ref, o2_ref, o3_ref, o4_ref, *, H, W):
    HW = H * W
    C = 128

    xb = x_ref[0].astype(jnp.bfloat16)
    fused = lax.dot_general(xb, fw_ref[...], (((0,), (0,)), ((), ())),
                            preferred_element_type=jnp.float32)
    fb = jnp.maximum(fused + fs_ref[...], ff_ref[...]).astype(jnp.bfloat16)

    def conv3(src, w_ref, s_ref):
        x3 = src.reshape(H, W, C)
        zr = jnp.zeros((1, W, C), jnp.bfloat16)
        xv = jnp.concatenate([zr, x3, zr], axis=0)
        rows3 = jnp.concatenate([xv[0:H], xv[1:H + 1], xv[2:H + 2]],
                                axis=-1).reshape(HW, 3 * C)
        z = jnp.dot(rows3, w_ref[...], preferred_element_type=jnp.float32)
        z0 = z[:, 0:C].reshape(H, W, C)
        z1 = z[:, C:2 * C].reshape(H, W, C)
        z2 = z[:, 2 * C:3 * C].reshape(H, W, C)
        zc = jnp.zeros((H, 1, C), jnp.float32)
        y = (z1 + jnp.concatenate([zc, z0[:, 0:W - 1]], axis=1)
             + jnp.concatenate([z2[:, 1:W], zc], axis=1)).reshape(HW, C)
        return jnp.maximum(y + s_ref[...], 0.0)

    x2 = conv3(fb[:, C:2 * C], w2_ref, s2_ref)
    t3 = conv3(fb[:, 2 * C:3 * C], w32_ref, s32_ref).astype(jnp.bfloat16)
    x3 = conv3(t3, w33_ref, s33_ref)

    f4 = fb[:, 3 * C:4 * C].astype(jnp.float32).reshape(H, W, C)
    zr = jnp.zeros((1, W, C), jnp.float32)
    xv = jnp.concatenate([zr, f4, zr], axis=0)
    rows = xv[0:H] + xv[1:H + 1] + xv[2:H + 2]
    zc = jnp.zeros((H, 1, C), jnp.float32)
    rp = jnp.concatenate([zc, rows, zc], axis=1)
    x4 = jnp.maximum((rp[:, 0:W] + rp[:, 1:W + 1] + rp[:, 2:W + 2])
                     .reshape(HW, C) + b4s_ref[...], 0.0)

    o1_ref[0] = fb[:, 0:C]
    o2_ref[0] = x2.astype(jnp.bfloat16)
    o3_ref[0] = x3.astype(jnp.bfloat16)
    o4_ref[0] = x4.astype(jnp.bfloat16)


def kernel(x_nchw, fused_w, fused_s, fused_floor, b4_s,
           b2_2_w, b2_2_s, b3_2_w, b3_2_s, b3_3_w, b3_3_s):
    N, Cin, H, W = x_nchw.shape
    HW = H * W
    x = x_nchw.reshape(N, Cin, HW)
    Cout = fused_w.shape[1]

    def _retap(w):
        C = w.shape[1]
        return w.reshape(3, 3, C, C).transpose(0, 2, 1, 3).reshape(3 * C, 3 * C)

    w2 = _retap(b2_2_w)
    w32 = _retap(b3_2_w)
    w33 = _retap(b3_3_w)

    obs = pl.BlockSpec((1, HW, 128), lambda n: (n, 0, 0))
    osh = jax.ShapeDtypeStruct((N, HW, 128), jnp.bfloat16)
    o1, o2, o3, o4 = pl.pallas_call(
        partial(_inception_kernel, H=H, W=W),
        out_shape=(osh, osh, osh, osh),
        grid=(N,),
        in_specs=[
            pl.BlockSpec((1, Cin, HW), lambda n: (n, 0, 0)),
            pl.BlockSpec((Cin, Cout), lambda n: (0, 0)),
            pl.BlockSpec((1, Cout), lambda n: (0, 0)),
            pl.BlockSpec((1, Cout), lambda n: (0, 0)),
            pl.BlockSpec((1, 128), lambda n: (0, 0)),
            pl.BlockSpec((384, 384), lambda n: (0, 0)),
            pl.BlockSpec((1, 128), lambda n: (0, 0)),
            pl.BlockSpec((384, 384), lambda n: (0, 0)),
            pl.BlockSpec((1, 128), lambda n: (0, 0)),
            pl.BlockSpec((384, 384), lambda n: (0, 0)),
            pl.BlockSpec((1, 128), lambda n: (0, 0)),
        ],
        out_specs=(obs, obs, obs, obs),
        compiler_params=pltpu.CompilerParams(
            dimension_semantics=("parallel",),
            vmem_limit_bytes=24 << 20),
    )(x, fused_w, fused_s, fused_floor, b4_s,
      w2, b2_2_s, w32, b3_2_s, w33, b3_3_s)

    def _tr(o):
        return jnp.transpose(o[:, :, 0:96], (0, 2, 1)).astype(jnp.float32)

    out = jnp.concatenate([_tr(o1), _tr(o2), _tr(o3), _tr(o4)], axis=1)
    return out.reshape(N, 384, H, W)
